# baseline (speedup 1.0000x reference)
import jax
import jax.numpy as jnp
from jax import lax
from jax.experimental import pallas as pl
from jax.experimental.pallas import tpu as pltpu
from jax.experimental.pallas import tpu_sc as plsc

_BATCH = 4096
_HIST = 200
_D = 64
_DP = 128
_V2 = 500000
_N = _BATCH * _HIST
_NC = 2
_NS = 16
_NW = _NC * _NS
_NPW = _N // _NW
_NBUF = 2
_C = 160
_G = _NPW // _C


def _body(x_hbm, table_hbm, out_hbm, idx_v, vps, rows, compacts, gsems, wsems):
    wid = lax.axis_index("s") * _NC + lax.axis_index("c")
    base = wid * _NPW
    pltpu.sync_copy(x_hbm.at[pl.ds(base, _NPW)], idx_v)

    def _gather_start(g, b):
        @pl.loop(0, _C, step=16)
        def _(i):
            vps[b][pl.ds(i, 16)] = lax.shift_right_logical(
                idx_v[pl.ds(g * _C + i, 16)], 1
            )
        pltpu.async_copy(table_hbm.at[vps[b]], rows[b], gsems[b])

    def _compact(g, b):
        @pl.loop(0, _C, step=16)
        def _(r0):
            vo = (idx_v[pl.ds(g * _C + r0, 16)] & 1) * _D
            for u in range(16):
                r = r0 + u
                off = vo[u]
                for k in range(_D // 16):
                    compacts[b][r, pl.ds(k * 16, 16)] = rows[b][
                        r, pl.ds(off + k * 16, 16)
                    ]

    for b in range(_NBUF):
        _gather_start(b, b)

    @pl.loop(0, _G, step=_NBUF)
    def _outer(g0):
        for b in range(_NBUF):
            g = g0 + b
            pltpu.make_async_copy(
                table_hbm.at[vps[b]], rows[b], gsems[b]
            ).wait()
            @pl.when(g >= _NBUF)
            def _():
                pltpu.make_async_copy(
                    compacts[b], out_hbm.at[pl.ds(0, _C)], wsems[b]
                ).wait()
            _compact(g, b)
            pltpu.async_copy(
                compacts[b], out_hbm.at[pl.ds(base + g * _C, _C)], wsems[b]
            )
            @pl.when(g + _NBUF < _G)
            def _():
                _gather_start(g + _NBUF, b)

    for b in range(_NBUF):
        pltpu.make_async_copy(
            compacts[b], out_hbm.at[pl.ds(0, _C)], wsems[b]
        ).wait()


@jax.jit
def _lookup(x_flat, table2):
    mesh = plsc.VectorSubcoreMesh(core_axis_name="c", subcore_axis_name="s")
    return pl.kernel(
        _body,
        out_type=jax.ShapeDtypeStruct((_N, _D), jnp.float32),
        mesh=mesh,
        scratch_types=[
            pltpu.VMEM((_NPW,), jnp.int32),
            [pltpu.VMEM((_C,), jnp.int32) for _ in range(_NBUF)],
            [pltpu.VMEM((_C, _DP), jnp.float32) for _ in range(_NBUF)],
            [pltpu.VMEM((_C, _D), jnp.float32) for _ in range(_NBUF)],
            [pltpu.SemaphoreType.DMA for _ in range(_NBUF)],
            [pltpu.SemaphoreType.DMA for _ in range(_NBUF)],
        ],
        compiler_params=pltpu.CompilerParams(use_tc_tiling_on_sc=True),
    )(x_flat, table2)


def kernel(x, table):
    table2 = table.reshape(_V2, _DP)
    out = _lookup(x.reshape(-1), table2)
    return out.reshape(_BATCH, _HIST, _D)


# speedup vs baseline: 1.1542x; 1.1542x over previous
import jax
import jax.numpy as jnp
from jax import lax
from jax.experimental import pallas as pl
from jax.experimental.pallas import tpu as pltpu
from jax.experimental.pallas import tpu_sc as plsc

_BATCH = 4096
_HIST = 200
_D = 64
_DP = 128
_N = _BATCH * _HIST
_NC = 2
_NS = 16
_NW = _NC * _NS
_NPW = _N // _NW
_NBUF = 4
_C = 200
_G = _NPW // _C


def _body(x_hbm, table_hbm, out_hbm, idx_v, rows, gsems, wsems):
    wid = lax.axis_index("s") * _NC + lax.axis_index("c")
    base = wid * _NPW
    pltpu.sync_copy(x_hbm.at[pl.ds(base, _NPW)], idx_v)

    def _gather_start(g, b):
        pltpu.async_copy(
            table_hbm.at[idx_v.at[pl.ds(g * _C, _C)]], rows[b], gsems[b]
        )

    for b in range(_NBUF):
        _gather_start(b, b)

    @pl.loop(0, _G, step=_NBUF)
    def _outer(g0):
        for b in range(_NBUF):
            g = g0 + b
            pltpu.make_async_copy(
                table_hbm.at[idx_v.at[pl.ds(g * _C, _C)]], rows[b], gsems[b]
            ).wait()
            pltpu.async_copy(
                rows[b], out_hbm.at[pl.ds(base + g * _C, _C)], wsems[b]
            )
            @pl.when(g + _NBUF < _G)
            def _():
                pltpu.make_async_copy(
                    rows[b], out_hbm.at[pl.ds(base + g * _C, _C)], wsems[b]
                ).wait()
                _gather_start(g + _NBUF, b)

    for b in range(_NBUF):
        g_last = _G - _NBUF + b
        pltpu.make_async_copy(
            rows[b], out_hbm.at[pl.ds(base + g_last * _C, _C)], wsems[b]
        ).wait()


@jax.jit
def _lookup(x_flat, table_pad):
    mesh = plsc.VectorSubcoreMesh(core_axis_name="c", subcore_axis_name="s")
    return pl.kernel(
        _body,
        out_type=jax.ShapeDtypeStruct((_N, _DP), jnp.float32),
        mesh=mesh,
        scratch_types=[
            pltpu.VMEM((_NPW,), jnp.int32),
            [pltpu.VMEM((_C, _DP), jnp.float32) for _ in range(_NBUF)],
            [pltpu.SemaphoreType.DMA for _ in range(_NBUF)],
            [pltpu.SemaphoreType.DMA for _ in range(_NBUF)],
        ],
        compiler_params=pltpu.CompilerParams(use_tc_tiling_on_sc=True),
    )(x_flat, table_pad)


def kernel(x, table):
    table_pad = jnp.pad(table.T, ((0, _DP - _D), (0, 0))).T
    out = _lookup(x.reshape(-1), table_pad)
    return out.reshape(_BATCH, _HIST, _DP)[:, :, :_D]


# speedup vs baseline: 1.1604x; 1.0054x over previous
import jax
import jax.numpy as jnp
from jax import lax
from jax.experimental import pallas as pl
from jax.experimental.pallas import tpu as pltpu
from jax.experimental.pallas import tpu_sc as plsc

_BATCH = 4096
_HIST = 200
_D = 64
_DP = 128
_NC = 2
_NS = 16
_NW = _NC * _NS
_BW = _BATCH // _NW
_NBUF = 4
_G = _HIST


def _body(xt_hbm, table_hbm, out_hbm, idx_v, rows, gsems, wsems):
    wid = lax.axis_index("s") * _NC + lax.axis_index("c")
    b0 = wid * _BW
    pltpu.sync_copy(xt_hbm.at[:, pl.ds(b0, _BW)], idx_v)

    def _gather_start(g, b):
        pltpu.async_copy(table_hbm.at[idx_v.at[g]], rows[b], gsems[b])

    for b in range(_NBUF):
        _gather_start(b, b)

    @pl.loop(0, _G, step=_NBUF)
    def _outer(g0):
        for b in range(_NBUF):
            g = g0 + b
            pltpu.make_async_copy(
                table_hbm.at[idx_v.at[g]], rows[b], gsems[b]
            ).wait()
            pltpu.async_copy(
                rows[b], out_hbm.at[pl.ds(b0, _BW), g, :], wsems[b]
            )
            @pl.when(g + _NBUF < _G)
            def _():
                pltpu.make_async_copy(
                    rows[b], out_hbm.at[pl.ds(b0, _BW), g, :], wsems[b]
                ).wait()
                _gather_start(g + _NBUF, b)

    for b in range(_NBUF):
        g_last = _G - _NBUF + b
        pltpu.make_async_copy(
            rows[b], out_hbm.at[pl.ds(b0, _BW), g_last, :], wsems[b]
        ).wait()


@jax.jit
def _lookup(xt, table_pad):
    mesh = plsc.VectorSubcoreMesh(core_axis_name="c", subcore_axis_name="s")
    return pl.kernel(
        _body,
        out_type=jax.ShapeDtypeStruct((_BATCH, _HIST, _DP), jnp.float32),
        mesh=mesh,
        scratch_types=[
            pltpu.VMEM((_HIST, _BW), jnp.int32),
            [pltpu.VMEM((_BW, _DP), jnp.float32) for _ in range(_NBUF)],
            [pltpu.SemaphoreType.DMA for _ in range(_NBUF)],
            [pltpu.SemaphoreType.DMA for _ in range(_NBUF)],
        ],
        compiler_params=pltpu.CompilerParams(use_tc_tiling_on_sc=True),
    )(xt, table_pad)


def kernel(x, table):
    table_pad = jnp.pad(table.T, ((0, _DP - _D), (0, 0))).T
    out = _lookup(x.T, table_pad)
    return out[:, :, :_D]
